# baseline (speedup 1.0000x reference)
import functools

import jax
import jax.numpy as jnp
from jax import lax
from jax.experimental import pallas as pl
from jax.experimental.pallas import tpu as pltpu
from jax.experimental.pallas import tpu_sc as plsc

_VOCAB = 1000000
_PREFIX = 100000
_SUFFIX = 100000
_EMB = 64
_WIN = 5
_HID = 512
_OUT = 50
_B = 16384

_NFLAT = _B * _WIN
_NW = 32
_PER_W = _NFLAT // _NW
_CHUNK = 256
_NCHUNK = _PER_W // _CHUNK

_sc_mesh = plsc.VectorSubcoreMesh(core_axis_name="c", subcore_axis_name="s")


@functools.partial(
    pl.kernel,
    mesh=_sc_mesh,
    compiler_params=pltpu.CompilerParams(use_tc_tiling_on_sc=False),
    out_type=jax.ShapeDtypeStruct((_NFLAT, _EMB), jnp.float32),
    scratch_types=[
        pltpu.VMEM((_PER_W,), jnp.int32),
        pltpu.VMEM((_PER_W,), jnp.int32),
        pltpu.VMEM((_PER_W,), jnp.int32),
        pltpu.VMEM((2, _CHUNK, _EMB), jnp.float32),
        pltpu.VMEM((2, _CHUNK, _EMB), jnp.float32),
        pltpu.VMEM((2, _CHUNK, _EMB), jnp.float32),
        pltpu.SemaphoreType.DMA,
        pltpu.SemaphoreType.DMA,
        pltpu.SemaphoreType.DMA,
        pltpu.SemaphoreType.DMA,
        pltpu.SemaphoreType.DMA,
        pltpu.SemaphoreType.DMA,
    ],
)
def _gather_sum(wemb, pemb, semb, idxw, idxp, idxs, out,
                idxw_v, idxp_v, idxs_v, rw, rp, rs,
                sw0, sw1, sp0, sp1, ss0, ss1):
    wid = lax.axis_index("s") * 2 + lax.axis_index("c")
    base = wid * _PER_W
    pltpu.sync_copy(idxw.at[pl.ds(base, _PER_W)], idxw_v)
    pltpu.sync_copy(idxp.at[pl.ds(base, _PER_W)], idxp_v)
    pltpu.sync_copy(idxs.at[pl.ds(base, _PER_W)], idxs_v)

    sems_w = (sw0, sw1)
    sems_p = (sp0, sp1)
    sems_s = (ss0, ss1)

    def start(c, buf):
        off = c * _CHUNK
        cw = pltpu.async_copy(
            wemb.at[idxw_v.at[pl.ds(off, _CHUNK)]], rw.at[buf], sems_w[buf])
        cp = pltpu.async_copy(
            pemb.at[idxp_v.at[pl.ds(off, _CHUNK)]], rp.at[buf], sems_p[buf])
        cs = pltpu.async_copy(
            semb.at[idxs_v.at[pl.ds(off, _CHUNK)]], rs.at[buf], sems_s[buf])
        return cw, cp, cs

    pending = start(0, 0)
    for c in range(_NCHUNK):
        buf = c & 1
        cur = pending
        if c + 1 < _NCHUNK:
            pending = start(c + 1, (c + 1) & 1)
        for h in cur:
            h.wait()

        def row_body(i, _):
            for j in range(_EMB // 16):
                sl = pl.ds(j * 16, 16)
                rw[buf, i, sl] = rw[buf, i, sl] + rp[buf, i, sl] + rs[buf, i, sl]
            return 0

        lax.fori_loop(0, _CHUNK, row_body, 0, unroll=2)
        pltpu.sync_copy(rw.at[buf], out.at[pl.ds(base + c * _CHUNK, _CHUNK)])


def _mlp_body(x_ref, w1_ref, b1_ref, w2_ref, b2_ref, o_ref):
    h = jnp.tanh(
        jnp.dot(x_ref[...], w1_ref[...], preferred_element_type=jnp.float32)
        + b1_ref[...])
    o_ref[...] = (
        jnp.dot(h, w2_ref[...], preferred_element_type=jnp.float32)
        + b2_ref[...])


_BM = 2048

_mlp = pl.pallas_call(
    _mlp_body,
    grid=(_B // _BM,),
    in_specs=[
        pl.BlockSpec((_BM, _WIN * _EMB), lambda i: (i, 0)),
        pl.BlockSpec((_WIN * _EMB, _HID), lambda i: (0, 0)),
        pl.BlockSpec((1, _HID), lambda i: (0, 0)),
        pl.BlockSpec((_HID, _OUT), lambda i: (0, 0)),
        pl.BlockSpec((1, _OUT), lambda i: (0, 0)),
    ],
    out_specs=pl.BlockSpec((_BM, _OUT), lambda i: (i, 0)),
    out_shape=jax.ShapeDtypeStruct((_B, _OUT), jnp.float32),
)


def kernel(words, prefixes, suffixes, word_emb, prefix_emb, suffix_emb,
           W1, b1, W2, b2):
    combined = _gather_sum(word_emb, prefix_emb, suffix_emb,
                           words.reshape(-1), prefixes.reshape(-1),
                           suffixes.reshape(-1))
    x = combined.reshape(_B, _WIN * _EMB)
    return _mlp(x, W1, b1.reshape(1, _HID), W2, b2.reshape(1, _OUT))
